# baseline (speedup 1.0000x reference)
import functools

import jax
import jax.numpy as jnp
from jax import lax
from jax.experimental import pallas as pl
from jax.experimental.pallas import tpu as pltpu
from jax.experimental.pallas import tpu_sc as plsc

N_VOCAB = 100000
D_MODEL = 128
N_SEQ = 16384
SEQ_LEN = 50
PAIR = 2
N_PAIRS = N_SEQ // PAIR
NUM_WORKERS = 32
P_PER_WORKER = N_PAIRS // NUM_WORKERS
NBUF = 4
LOOKAHEAD = NBUF - 1


def _gather_kernel(idx_hbm, table_hbm, out_hbm,
                   idx_v, b0, b1, b2, b3, g0, g1, g2, g3, s0, s1, s2, s3):
    wid = lax.axis_index("s") * 2 + lax.axis_index("c")
    base = wid * P_PER_WORKER
    pltpu.sync_copy(idx_hbm.at[pl.ds(base, P_PER_WORKER)], idx_v)

    bufs = (b0, b1, b2, b3)
    gsems = (g0, g1, g2, g3)
    ssems = (s0, s1, s2, s3)

    def gather_desc(p, slot):
        return pltpu.make_async_copy(table_hbm.at[idx_v.at[p]],
                                     bufs[slot], gsems[slot])

    def store_desc(p, slot):
        return pltpu.make_async_copy(bufs[slot].reshape(PAIR, SEQ_LEN, D_MODEL),
                                     out_hbm.at[pl.ds((base + p) * PAIR, PAIR)],
                                     ssems[slot])

    for c in range(LOOKAHEAD):
        gather_desc(c, c).start()

    def body(o, carry):
        for b in range(NBUF):
            t = NBUF * o + b
            gather_desc(t, b).wait()
            store_desc(t, b).start()
            f = t + LOOKAHEAD
            fslot = (b + LOOKAHEAD) % NBUF

            @pl.when(jnp.logical_and(f < P_PER_WORKER, t >= 1))
            def _():
                store_desc(t - 1, fslot).wait()

            @pl.when(f < P_PER_WORKER)
            def _():
                gather_desc(f, fslot).start()
        return carry

    lax.fori_loop(0, P_PER_WORKER // NBUF, body, 0)

    for c in range(P_PER_WORKER - NBUF, P_PER_WORKER):
        store_desc(c, c % NBUF).wait()


def kernel(inputs, embedding_weight):
    idx = inputs.reshape(N_PAIRS, PAIR * SEQ_LEN).astype(jnp.int32)
    mesh = plsc.VectorSubcoreMesh(core_axis_name="c", subcore_axis_name="s")
    run = functools.partial(
        pl.kernel,
        mesh=mesh,
        out_type=jax.ShapeDtypeStruct((N_SEQ, SEQ_LEN, D_MODEL), jnp.float32),
        compiler_params=pltpu.CompilerParams(needs_layout_passes=True, use_tc_tiling_on_sc=True),
        scratch_types=(
            [pltpu.VMEM((P_PER_WORKER, PAIR * SEQ_LEN), jnp.int32)]
            + [pltpu.VMEM((PAIR * SEQ_LEN, D_MODEL), jnp.float32)] * NBUF
            + [pltpu.SemaphoreType.DMA] * (2 * NBUF)
        ),
    )(_gather_kernel)
    return run(idx, embedding_weight)


# speedup vs baseline: 1.8833x; 1.8833x over previous
import functools

import jax
import jax.numpy as jnp
from jax import lax
from jax.experimental import pallas as pl
from jax.experimental.pallas import tpu as pltpu
from jax.experimental.pallas import tpu_sc as plsc

N_VOCAB = 100000
D_MODEL = 128
N_SEQ = 16384
SEQ_LEN = 50
B_ROWS = N_SEQ * SEQ_LEN
NUM_WORKERS = 32
ROWS_PER_WORKER = B_ROWS // NUM_WORKERS
G = 128
NCH = ROWS_PER_WORKER // G
NBUF = 4
LOOKAHEAD = NBUF - 1


def _gather_kernel(idx_hbm, table_hbm, out_hbm,
                   idx_v, b0, b1, b2, b3, g0, g1, g2, g3, s0, s1, s2, s3):
    wid = lax.axis_index("s") * 2 + lax.axis_index("c")
    base = wid * NCH
    pltpu.sync_copy(idx_hbm.at[pl.ds(base, NCH)], idx_v)

    bufs = (b0, b1, b2, b3)
    gsems = (g0, g1, g2, g3)
    ssems = (s0, s1, s2, s3)

    def gather_desc(chunk, slot):
        return pltpu.make_async_copy(table_hbm.at[idx_v.at[chunk]],
                                     bufs[slot], gsems[slot])

    def store_desc(chunk, slot):
        return pltpu.make_async_copy(bufs[slot],
                                     out_hbm.at[pl.ds((base + chunk) * G, G)],
                                     ssems[slot])

    for c in range(LOOKAHEAD):
        gather_desc(c, c).start()

    def body(o, carry):
        for b in range(NBUF):
            t = NBUF * o + b
            gather_desc(t, b).wait()
            store_desc(t, b).start()
            f = t + LOOKAHEAD
            fslot = (b + LOOKAHEAD) % NBUF

            @pl.when(jnp.logical_and(f < NCH, t >= 1))
            def _():
                store_desc(t - 1, fslot).wait()

            @pl.when(f < NCH)
            def _():
                gather_desc(f, fslot).start()
        return carry

    lax.fori_loop(0, NCH // NBUF, body, 0)

    for c in range(NCH - NBUF, NCH):
        store_desc(c, c % NBUF).wait()


def kernel(inputs, embedding_weight):
    idx = inputs.astype(jnp.int32).T.reshape(B_ROWS // G, G)
    mesh = plsc.VectorSubcoreMesh(core_axis_name="c", subcore_axis_name="s")
    run = functools.partial(
        pl.kernel,
        mesh=mesh,
        out_type=jax.ShapeDtypeStruct((B_ROWS, D_MODEL), jnp.float32),
        scratch_types=(
            [pltpu.VMEM((NCH, G), jnp.int32)]
            + [pltpu.VMEM((G, D_MODEL), jnp.float32)] * NBUF
            + [pltpu.SemaphoreType.DMA] * (2 * NBUF)
        ),
    )(_gather_kernel)
    out = run(idx, embedding_weight)
    return out.reshape(SEQ_LEN, N_SEQ, D_MODEL).transpose(1, 0, 2)


# speedup vs baseline: 1.8889x; 1.0030x over previous
import functools

import jax
import jax.numpy as jnp
from jax import lax
from jax.experimental import pallas as pl
from jax.experimental.pallas import tpu as pltpu
from jax.experimental.pallas import tpu_sc as plsc

N_VOCAB = 100000
D_MODEL = 128
N_SEQ = 16384
SEQ_LEN = 50
B_ROWS = N_SEQ * SEQ_LEN
NUM_WORKERS = 32
ROWS_PER_WORKER = B_ROWS // NUM_WORKERS
G = 128
NCH = ROWS_PER_WORKER // G
NBUF = 5
LOOKAHEAD = NBUF - 1


def _gather_kernel(idx_hbm, table_hbm, out_hbm,
                   idx_v, b0, b1, b2, b3, b4,
                   g0, g1, g2, g3, g4, s0, s1, s2, s3, s4):
    wid = lax.axis_index("s") * 2 + lax.axis_index("c")
    base = wid * NCH
    pltpu.sync_copy(idx_hbm.at[pl.ds(base, NCH)], idx_v)

    bufs = (b0, b1, b2, b3, b4)
    gsems = (g0, g1, g2, g3, g4)
    ssems = (s0, s1, s2, s3, s4)

    def gather_desc(chunk, slot):
        return pltpu.make_async_copy(table_hbm.at[idx_v.at[chunk]],
                                     bufs[slot], gsems[slot])

    def store_desc(chunk, slot):
        return pltpu.make_async_copy(bufs[slot],
                                     out_hbm.at[pl.ds((base + chunk) * G, G)],
                                     ssems[slot])

    for c in range(LOOKAHEAD):
        gather_desc(c, c).start()

    def body(o, carry):
        for b in range(NBUF):
            t = NBUF * o + b
            gather_desc(t, b).wait()
            store_desc(t, b).start()
            f = t + LOOKAHEAD
            fslot = (b + LOOKAHEAD) % NBUF

            @pl.when(jnp.logical_and(f < NCH, t >= 1))
            def _():
                store_desc(t - 1, fslot).wait()

            @pl.when(f < NCH)
            def _():
                gather_desc(f, fslot).start()
        return carry

    lax.fori_loop(0, NCH // NBUF, body, 0)

    for c in range(NCH - NBUF, NCH):
        store_desc(c, c % NBUF).wait()


def kernel(inputs, embedding_weight):
    idx = inputs.astype(jnp.int32).T.reshape(B_ROWS // G, G)
    mesh = plsc.VectorSubcoreMesh(core_axis_name="c", subcore_axis_name="s")
    run = functools.partial(
        pl.kernel,
        mesh=mesh,
        out_type=jax.ShapeDtypeStruct((B_ROWS, D_MODEL), jnp.float32),
        scratch_types=(
            [pltpu.VMEM((NCH, G), jnp.int32)]
            + [pltpu.VMEM((G, D_MODEL), jnp.float32)] * NBUF
            + [pltpu.SemaphoreType.DMA] * (2 * NBUF)
        ),
    )(_gather_kernel)
    out = run(idx, embedding_weight)
    return out.reshape(SEQ_LEN, N_SEQ, D_MODEL).transpose(1, 0, 2)
